# baseline (speedup 1.0000x reference)
import functools

import jax
import jax.numpy as jnp
from jax import lax
from jax.experimental import pallas as pl
from jax.experimental.pallas import tpu as pltpu
from jax.experimental.pallas import tpu_sc as plsc

_N = 10000
_NC = 2
_NS = 16
_NW = _NC * _NS
_CH = 64
_NB = 4
_LA = 2
_NP = 10112
_FR0 = 0.42
_NPS = _NP // _NS


def _make_scatter(dd: int, nch0: int, nch1: int):
  nch = max(nch0, nch1)
  mesh = plsc.VectorSubcoreMesh(core_axis_name="c", subcore_axis_name="s")

  @functools.partial(
      pl.kernel,
      out_type=jax.ShapeDtypeStruct((_NC, _NP, dd), jnp.float32),
      mesh=mesh,
      compiler_params=pltpu.CompilerParams(use_tc_tiling_on_sc=(dd == 128)),
      scratch_types=[
          pltpu.VMEM((_NB, 3, _CH), jnp.int32),
          pltpu.VMEM((_NB, _CH), jnp.int32),
          pltpu.VMEM((_NB, _CH, dd), jnp.float32),
          pltpu.VMEM_SHARED((_NP, dd), jnp.float32),
      ] + [pltpu.SemaphoreType.DMA] * (3 * _NB),
  )
  def k(table, edges, zeros, parts, edge_v, cols_d, msg_v, acc, *sems):
    esem = sems[:_NB]
    gsem = sems[_NB:2 * _NB]
    ssem = sems[2 * _NB:]
    c = lax.axis_index("c")
    s = lax.axis_index("s")
    wid = c * _NS + s
    ncha = jnp.where(c == 0, nch0, nch1)
    pltpu.sync_copy(zeros.at[pl.ds(s * _NPS, _NPS)],
                    acc.at[pl.ds(s * _NPS, _NPS)])
    plsc.subcore_barrier()

    def fire_edge(j, b):
      pltpu.async_copy(edges.at[wid, j], edge_v.at[b], esem[b])

    def wait_edge(b):
      pltpu.make_async_copy(edges.at[wid, 0], edge_v.at[b], esem[b]).wait()

    def decode(b):
      for t in range(_CH // 16):
        sl = pl.ds(t * 16, 16)
        cols_d[b, sl] = edge_v[b, 1, sl]

    def fire_gather(b):
      pltpu.async_copy(table.at[edge_v.at[b, 0]], msg_v.at[b], gsem[b])

    def wait_gather(b):
      pltpu.make_async_copy(table.at[edge_v.at[0, 0]], msg_v.at[b],
                            gsem[b]).wait()

    def scale(b):
      def sbody(t, carry):
        wf = lax.bitcast_convert_type(edge_v[b, 2, pl.ds(t * 16, 16)], jnp.float32)
        for l in range(16):
          wl = wf[l]
          i = t * 16 + l
          for kk in range(dd // 16):
            sl = pl.ds(kk * 16, 16)
            msg_v[b, i, sl] = msg_v[b, i, sl] * wl
        return carry

      lax.fori_loop(0, _CH // 16, sbody, 0)

    def fire_scatter(b):
      pltpu.async_copy(msg_v.at[b], acc.at[cols_d.at[b]], ssem[b], add=True)

    def wait_scatter(b):
      pltpu.make_async_copy(msg_v.at[b], acc.at[cols_d.at[0]],
                            ssem[b]).wait()

    for j in range(_NB):
      fire_edge(j, j)
    for j in range(_LA):
      wait_edge(j)
      decode(j)
      fire_gather(j)
    for j in range(_NB - _LA):
      b2 = (j + _LA) % _NB
      wait_edge(b2)
      decode(b2)
      fire_gather(b2)
      wait_gather(j % _NB)
      scale(j % _NB)
      fire_scatter(j % _NB)
      fire_edge(j + _NB, j % _NB)

    def group(g, carry):
      for u in range(_NB):
        j = g * _NB + (_NB - _LA) + u
        bj = (u + _NB - _LA) % _NB
        b2 = u
        wait_scatter(b2)
        wait_edge(b2)
        decode(b2)
        fire_gather(b2)
        wait_gather(bj)
        scale(bj)
        fire_scatter(bj)

        @pl.when(j + _NB < ncha)
        def _():
          fire_edge(j + _NB, bj)

      return carry

    lax.fori_loop(0, (ncha - 2 * _LA) // _NB, group, 0)
    for k in range(_LA):
      wait_scatter(k % _NB)
      wait_gather((k + _NB - _LA) % _NB)
      scale((k + _NB - _LA) % _NB)
      fire_scatter((k + _NB - _LA) % _NB)
    for k in range(_LA):
      wait_scatter((k + _NB - _LA) % _NB)

    plsc.subcore_barrier()
    pltpu.sync_copy(acc.at[pl.ds(s * _NPS, _NPS)],
                    parts.at[c, pl.ds(s * _NPS, _NPS)])

  return k


def _tc_mm1(x, W1, Wres):
  n, _ = x.shape
  h = W1.shape[1]

  def body(x_r, w1_r, wres_r, g1_r, res_r):
    g1_r[...] = jnp.dot(x_r[...], w1_r[...],
                        preferred_element_type=jnp.float32)
    res_r[...] = jnp.dot(x_r[...], wres_r[...],
                         preferred_element_type=jnp.float32)

  return pl.pallas_call(
      body,
      out_shape=[
          jax.ShapeDtypeStruct((n, h), jnp.float32),
          jax.ShapeDtypeStruct((n, h), jnp.float32),
      ],
  )(x, W1, Wres)


def _tc_dinv(dparts, g1):
  n, h = g1.shape

  def body(dp_r, g1_r, dinv_r, g1p_r):
    deg = 1.0 + dp_r[0, :, 0:1] + dp_r[1, :, 0:1]
    dinv = jnp.where(deg > 0, lax.rsqrt(jnp.maximum(deg, 1e-12)), 0.0)
    dinv_r[...] = dinv
    g1p_r[...] = g1_r[...] * dinv

  return pl.pallas_call(
      body,
      out_shape=[
          jax.ShapeDtypeStruct((n, 1), jnp.float32),
          jax.ShapeDtypeStruct((n, h), jnp.float32),
      ],
  )(dparts, g1)


def _tc_mid(parts, g, dinv, b, gamma, beta, Wn, res=None):
  n, hh = g.shape
  ho = Wn.shape[1]
  has_res = res is not None

  def body(*refs):
    if has_res:
      (p_r, g_r, dinv_r, b_r, ga_r, be_r, wn_r, res_r, gn_r, gnp_r) = refs
    else:
      (p_r, g_r, dinv_r, b_r, ga_r, be_r, wn_r, gn_r, gnp_r) = refs
    dinv = dinv_r[...]
    t = dinv * (p_r[0] + p_r[1]) + dinv * dinv * g_r[...] + b_r[...]
    if has_res:
      t = t + res_r[...]
    mu = jnp.mean(t, axis=0, keepdims=True)
    var = jnp.mean((t - mu) ** 2, axis=0, keepdims=True)
    hact = ga_r[...] * (t - mu) / jnp.sqrt(var + 1e-5) + be_r[...]
    hact = jnp.maximum(hact, 0.0)
    gn = jnp.dot(hact, wn_r[...], preferred_element_type=jnp.float32)
    gn_r[...] = gn
    gnp_r[...] = gn * dinv

  args = [parts, g, dinv, b.reshape(1, hh), gamma.reshape(1, hh),
          beta.reshape(1, hh), Wn]
  if has_res:
    args.append(res)
  return pl.pallas_call(
      body,
      out_shape=[
          jax.ShapeDtypeStruct((n, ho), jnp.float32),
          jax.ShapeDtypeStruct((n, ho), jnp.float32),
      ],
  )(*args)


def _tc_pad16(gn, gnp):
  n = gn.shape[0]

  def body(gnp_r, out_r):
    colmask = lax.broadcasted_iota(jnp.int32, (n, 16), 1) == 0
    out_r[...] = jnp.where(colmask, gnp_r[...], 0.0)

  return pl.pallas_call(
      body, out_shape=jax.ShapeDtypeStruct((n, 16), jnp.float32))(gnp)


def _tc_final(parts, g3, dinv, b3):
  n = g3.shape[0]

  def body(p_r, g3_r, dinv_r, b3_r, out_r):
    dinv = dinv_r[...]
    t = dinv * (p_r[0, :, 0:1] + p_r[1, :, 0:1])
    out_r[...] = t + dinv * dinv * g3_r[...] + b3_r[...]

  return pl.pallas_call(
      body, out_shape=jax.ShapeDtypeStruct((n, 1), jnp.float32))(
          parts, g3, dinv, b3.reshape(1, 1))


def kernel(x, edge_index, edge_weight, W1, b1, g1, be1, W2, b2, g2, be2, W3,
           b3, Wres):
  n, d = x.shape
  e = edge_index.shape[1]
  t = -(-e // (_NS * _CH))
  nch0 = max(8, 4 * round(t * _FR0 / 4))
  nch1 = -(-(t - nch0) // 4) * 4
  ep = _NS * (nch0 + nch1) * _CH
  pad = ep - e
  ncm = max(nch0, nch1)

  def _part(a):
    a0 = a[:_NS * nch0 * _CH].reshape(_NS, nch0, 1, _CH)
    a1 = a[_NS * nch0 * _CH:].reshape(_NS, nch1, 1, _CH)
    a0 = jnp.pad(a0, ((0, 0), (0, ncm - nch0), (0, 0), (0, 0)))
    a1 = jnp.pad(a1, ((0, 0), (0, ncm - nch1), (0, 0), (0, 0)))
    return jnp.concatenate([a0, a1], axis=0)

  rowp = jnp.pad(edge_index[0], (0, pad))
  colp = jnp.pad(edge_index[1], (0, pad))
  wp = jnp.pad(edge_weight, (0, pad))
  wbits = lax.bitcast_convert_type(wp, jnp.int32)
  edges = jnp.concatenate([_part(rowp), _part(colp), _part(wbits)], axis=2)

  ones16 = jnp.ones((n, 16), jnp.float32)
  zeros16 = jnp.zeros((_NP, 16), jnp.float32)
  zerosd = jnp.zeros((_NP, d), jnp.float32)

  scat16 = _make_scatter(16, nch0, nch1)
  scatd = _make_scatter(d, nch0, nch1)

  dparts = scat16(ones16, edges, zeros16)[:, :n]
  g1v, res = _tc_mm1(x, W1, Wres)
  dinv, g1p = _tc_dinv(dparts, g1v)

  parts1 = scatd(g1p, edges, zerosd)[:, :n]
  g2v, g2p = _tc_mid(parts1, g1v, dinv, b1, g1, be1, W2)

  parts2 = scatd(g2p, edges, zerosd)[:, :n]
  g3v, g3p = _tc_mid(parts2, g2v, dinv, b2, g2, be2, W3, res=res)

  g3pad = _tc_pad16(g3v, g3p)
  parts3 = scat16(g3pad, edges, zeros16)[:, :n]
  out = _tc_final(parts3, g3v, dinv, b3)
  return out[:, 0]


# speedup vs baseline: 1.0549x; 1.0549x over previous
import functools

import jax
import jax.numpy as jnp
from jax import lax
from jax.experimental import pallas as pl
from jax.experimental.pallas import tpu as pltpu
from jax.experimental.pallas import tpu_sc as plsc

_N = 10000
_NC = 2
_NS = 16
_NW = _NC * _NS
_CH = 64
_NB = 4
_LA = 2
_NP = 10112
_FR0 = 0.375
_NPS = _NP // _NS


def _make_scatter(dd: int, nch0: int, nch1: int):
  nch = max(nch0, nch1)
  mesh = plsc.VectorSubcoreMesh(core_axis_name="c", subcore_axis_name="s")

  @functools.partial(
      pl.kernel,
      out_type=jax.ShapeDtypeStruct((_NC, _NP, dd), jnp.float32),
      mesh=mesh,
      compiler_params=pltpu.CompilerParams(use_tc_tiling_on_sc=(dd == 128)),
      scratch_types=[
          pltpu.VMEM((_NB, 3, _CH), jnp.int32),
          pltpu.VMEM((_NB, _CH), jnp.int32),
          pltpu.VMEM((_NB, _CH, dd), jnp.float32),
          pltpu.VMEM_SHARED((_NP, dd), jnp.float32),
      ] + [pltpu.SemaphoreType.DMA] * (3 * _NB),
  )
  def k(table, edges, zeros, parts, edge_v, cols_d, msg_v, acc, *sems):
    esem = sems[:_NB]
    gsem = sems[_NB:2 * _NB]
    ssem = sems[2 * _NB:]
    c = lax.axis_index("c")
    s = lax.axis_index("s")
    wid = c * _NS + s
    ncha = jnp.where(c == 0, nch0, nch1)
    pltpu.sync_copy(zeros.at[pl.ds(s * _NPS, _NPS)],
                    acc.at[pl.ds(s * _NPS, _NPS)])
    plsc.subcore_barrier()

    def fire_edge(j, b):
      pltpu.async_copy(edges.at[wid, j], edge_v.at[b], esem[b])

    def wait_edge(b):
      pltpu.make_async_copy(edges.at[wid, 0], edge_v.at[b], esem[b]).wait()

    def decode(b):
      for t in range(_CH // 16):
        sl = pl.ds(t * 16, 16)
        cols_d[b, sl] = edge_v[b, 1, sl]

    def fire_gather(b):
      pltpu.async_copy(table.at[edge_v.at[b, 0]], msg_v.at[b], gsem[b])

    def wait_gather(b):
      pltpu.make_async_copy(table.at[edge_v.at[0, 0]], msg_v.at[b],
                            gsem[b]).wait()

    def scale(b):
      def sbody(t, carry):
        wf = lax.bitcast_convert_type(edge_v[b, 2, pl.ds(t * 16, 16)], jnp.float32)
        for l in range(16):
          wl = wf[l]
          i = t * 16 + l
          for kk in range(dd // 16):
            sl = pl.ds(kk * 16, 16)
            msg_v[b, i, sl] = msg_v[b, i, sl] * wl
        return carry

      lax.fori_loop(0, _CH // 16, sbody, 0)

    def fire_scatter(b):
      pltpu.async_copy(msg_v.at[b], acc.at[cols_d.at[b]], ssem[b], add=True)

    def wait_scatter(b):
      pltpu.make_async_copy(msg_v.at[b], acc.at[cols_d.at[0]],
                            ssem[b]).wait()

    for j in range(_NB):
      fire_edge(j, j)
    for j in range(_LA):
      wait_edge(j)
      decode(j)
      fire_gather(j)
    for j in range(_NB - _LA):
      b2 = (j + _LA) % _NB
      wait_edge(b2)
      decode(b2)
      fire_gather(b2)
      wait_gather(j % _NB)
      scale(j % _NB)
      fire_scatter(j % _NB)
      fire_edge(j + _NB, j % _NB)

    def group(g, carry):
      for u in range(_NB):
        j = g * _NB + (_NB - _LA) + u
        bj = (u + _NB - _LA) % _NB
        b2 = u
        wait_scatter(b2)
        wait_edge(b2)
        decode(b2)
        fire_gather(b2)
        wait_gather(bj)
        scale(bj)
        fire_scatter(bj)

        @pl.when(j + _NB < ncha)
        def _():
          fire_edge(j + _NB, bj)

      return carry

    lax.fori_loop(0, (ncha - 2 * _LA) // _NB, group, 0)
    for k in range(_LA):
      wait_scatter(k % _NB)
      wait_gather((k + _NB - _LA) % _NB)
      scale((k + _NB - _LA) % _NB)
      fire_scatter((k + _NB - _LA) % _NB)
    for k in range(_LA):
      wait_scatter((k + _NB - _LA) % _NB)

    plsc.subcore_barrier()
    pltpu.sync_copy(acc.at[pl.ds(s * _NPS, _NPS)],
                    parts.at[c, pl.ds(s * _NPS, _NPS)])

  return k


def _tc1(x, W1, Wres, dparts):
  n, _ = x.shape
  h = W1.shape[1]

  def body(x_r, w1_r, wres_r, dp_r, dinv_r, g1_r, g1p_r, res_r):
    deg = 1.0 + dp_r[0, :, 0:1] + dp_r[1, :, 0:1]
    dinv = jnp.where(deg > 0, lax.rsqrt(jnp.maximum(deg, 1e-12)), 0.0)
    dinv_r[...] = dinv
    g1 = jnp.dot(x_r[...], w1_r[...], preferred_element_type=jnp.float32)
    g1_r[...] = g1
    g1p_r[...] = g1 * dinv
    res_r[...] = jnp.dot(x_r[...], wres_r[...],
                         preferred_element_type=jnp.float32)

  return pl.pallas_call(
      body,
      out_shape=[
          jax.ShapeDtypeStruct((n, 1), jnp.float32),
          jax.ShapeDtypeStruct((n, h), jnp.float32),
          jax.ShapeDtypeStruct((n, h), jnp.float32),
          jax.ShapeDtypeStruct((n, h), jnp.float32),
      ],
  )(x, W1, Wres, dparts)


def _tc_mid(parts, g, dinv, b, gamma, beta, Wn, res=None):
  n, hh = g.shape
  ho = Wn.shape[1]
  has_res = res is not None

  def body(*refs):
    if has_res:
      (p_r, g_r, dinv_r, b_r, ga_r, be_r, wn_r, res_r, gn_r, gnp_r) = refs
    else:
      (p_r, g_r, dinv_r, b_r, ga_r, be_r, wn_r, gn_r, gnp_r) = refs
    dinv = dinv_r[...]
    t = dinv * (p_r[0] + p_r[1]) + dinv * dinv * g_r[...] + b_r[...]
    if has_res:
      t = t + res_r[...]
    mu = jnp.mean(t, axis=0, keepdims=True)
    var = jnp.mean((t - mu) ** 2, axis=0, keepdims=True)
    hact = ga_r[...] * (t - mu) / jnp.sqrt(var + 1e-5) + be_r[...]
    hact = jnp.maximum(hact, 0.0)
    gn = jnp.dot(hact, wn_r[...], preferred_element_type=jnp.float32)
    gn_r[...] = gn
    gnp_r[...] = gn * dinv

  args = [parts, g, dinv, b.reshape(1, hh), gamma.reshape(1, hh),
          beta.reshape(1, hh), Wn]
  if has_res:
    args.append(res)
  return pl.pallas_call(
      body,
      out_shape=[
          jax.ShapeDtypeStruct((n, ho), jnp.float32),
          jax.ShapeDtypeStruct((n, ho), jnp.float32),
      ],
  )(*args)


def _tc_pad16(gn, gnp):
  n = gn.shape[0]

  def body(gnp_r, out_r):
    colmask = lax.broadcasted_iota(jnp.int32, (n, 16), 1) == 0
    out_r[...] = jnp.where(colmask, gnp_r[...], 0.0)

  return pl.pallas_call(
      body, out_shape=jax.ShapeDtypeStruct((n, 16), jnp.float32))(gnp)


def _tc_final(parts, g3, dinv, b3):
  n = g3.shape[0]

  def body(p_r, g3_r, dinv_r, b3_r, out_r):
    dinv = dinv_r[...]
    t = dinv * (p_r[0, :, 0:1] + p_r[1, :, 0:1])
    out_r[...] = t + dinv * dinv * g3_r[...] + b3_r[...]

  return pl.pallas_call(
      body, out_shape=jax.ShapeDtypeStruct((n, 1), jnp.float32))(
          parts, g3, dinv, b3.reshape(1, 1))


def kernel(x, edge_index, edge_weight, W1, b1, g1, be1, W2, b2, g2, be2, W3,
           b3, Wres):
  n, d = x.shape
  e = edge_index.shape[1]
  t = -(-e // (_NS * _CH))
  nch0 = max(8, 4 * round(t * _FR0 / 4))
  nch1 = -(-(t - nch0) // 4) * 4
  ep = _NS * (nch0 + nch1) * _CH
  pad = ep - e
  ncm = max(nch0, nch1)

  def _part(a):
    a0 = a[:_NS * nch0 * _CH].reshape(_NS, nch0, 1, _CH)
    a1 = a[_NS * nch0 * _CH:].reshape(_NS, nch1, 1, _CH)
    a0 = jnp.pad(a0, ((0, 0), (0, ncm - nch0), (0, 0), (0, 0)))
    a1 = jnp.pad(a1, ((0, 0), (0, ncm - nch1), (0, 0), (0, 0)))
    return jnp.concatenate([a0, a1], axis=0)

  rowp = jnp.pad(edge_index[0], (0, pad))
  colp = jnp.pad(edge_index[1], (0, pad))
  wp = jnp.pad(edge_weight, (0, pad))
  wbits = lax.bitcast_convert_type(wp, jnp.int32)
  edges = jnp.concatenate([_part(rowp), _part(colp), _part(wbits)], axis=2)

  ones16 = jnp.ones((n, 16), jnp.float32)
  zeros16 = jnp.zeros((_NP, 16), jnp.float32)
  zerosd = jnp.zeros((_NP, d), jnp.float32)

  scat16 = _make_scatter(16, nch0, nch1)
  scatd = _make_scatter(d, nch0, nch1)

  dparts = scat16(ones16, edges, zeros16)[:, :n]
  dinv, g1v, g1p, res = _tc1(x, W1, Wres, dparts)

  parts1 = scatd(g1p, edges, zerosd)[:, :n]
  g2v, g2p = _tc_mid(parts1, g1v, dinv, b1, g1, be1, W2)

  parts2 = scatd(g2p, edges, zerosd)[:, :n]
  g3v, g3p = _tc_mid(parts2, g2v, dinv, b2, g2, be2, W3, res=res)

  g3pad = _tc_pad16(g3v, g3p)
  parts3 = scat16(g3pad, edges, zeros16)[:, :n]
  out = _tc_final(parts3, g3v, dinv, b3)
  return out[:, 0]
